# Initial kernel scaffold; baseline (speedup 1.0000x reference)
import jax
import jax.numpy as jnp
from jax.experimental import pallas as pl


def kernel(x, edge_index, W0, b0, W1, b1, W2, b2, lin1_W, lin1_b, lin2_W, lin2_b):
    raise NotImplementedError("write your pallas kernel here")


# speedup vs baseline: 7.3265x; 7.3265x over previous
import functools

import jax
import jax.numpy as jnp
from jax import lax
from jax.experimental import pallas as pl
from jax.experimental.pallas import tpu as pltpu
from jax.experimental.pallas import tpu_sc as plsc

N = 10000
E = 320000
D = 128
OUT = 40
NP = 10240
NC = 2
NS = 16
NW = NC * NS
CHUNK = 128
CPT = 80
EP = NW * CPT * CHUNK
DUMMY = N + 16
RPT = NP // NS
RB = 1024

_mesh = plsc.VectorSubcoreMesh(core_axis_name="c", subcore_axis_name="s")


def _hist_body(dst_hbm, out_hbm, idx_v, ones_v, zero_v, acc_sh):
    c = lax.axis_index("c")
    s = lax.axis_index("s")
    wid = c * NS + s

    @pl.loop(0, CHUNK)
    def _(i):
        ones_v[i, :] = jnp.full((16,), 1.0, jnp.float32)
        zero_v[i, :] = jnp.zeros((16,), jnp.float32)

    @pl.loop(0, RPT // CHUNK)
    def _(k):
        pltpu.sync_copy(zero_v, acc_sh.at[pl.ds(s * RPT + k * CHUNK, CHUNK)])

    pltpu.sync_copy(dst_hbm.at[wid], idx_v)
    plsc.subcore_barrier()

    @pl.loop(0, CPT)
    def _(j):
        pltpu.sync_copy(ones_v, acc_sh.at[idx_v.at[j]], add=True)

    plsc.subcore_barrier()

    @pl.loop(0, RPT // CHUNK)
    def _(k):
        row = s * RPT + k * CHUNK
        pltpu.sync_copy(acc_sh.at[pl.ds(row, CHUNK)], out_hbm.at[c, pl.ds(row, CHUNK)])


@jax.jit
def _hist(dstp):
    k = pl.kernel(
        _hist_body,
        out_type=jax.ShapeDtypeStruct((NC, NP, 16), jnp.float32),
        mesh=_mesh,
        scratch_types=[
            pltpu.VMEM((CPT, CHUNK), jnp.int32),
            pltpu.VMEM((CHUNK, 16), jnp.float32),
            pltpu.VMEM((CHUNK, 16), jnp.float32),
            pltpu.VMEM_SHARED((NP, 16), jnp.float32),
        ],
    )
    return k(dstp)


def _agg_body(hwp_hbm, src_hbm, dst_hbm, out_hbm, srcv, dstv, rows_v, acc_sh):
    c = lax.axis_index("c")
    s = lax.axis_index("s")
    wid = c * NS + s

    @pl.loop(0, CHUNK)
    def _(i):
        @pl.loop(0, D // 16)
        def _(j):
            rows_v[i, pl.ds(j * 16, 16)] = jnp.zeros((16,), jnp.float32)

    @pl.loop(0, RPT // CHUNK)
    def _(k):
        pltpu.sync_copy(rows_v, acc_sh.at[pl.ds(s * RPT + k * CHUNK, CHUNK)])

    pltpu.sync_copy(src_hbm.at[wid], srcv)
    pltpu.sync_copy(dst_hbm.at[wid], dstv)
    plsc.subcore_barrier()

    @pl.loop(0, CPT)
    def _(j):
        pltpu.sync_copy(hwp_hbm.at[srcv.at[j]], rows_v)
        pltpu.sync_copy(rows_v, acc_sh.at[dstv.at[j]], add=True)

    plsc.subcore_barrier()

    @pl.loop(0, RPT // CHUNK)
    def _(k):
        row = s * RPT + k * CHUNK
        pltpu.sync_copy(acc_sh.at[pl.ds(row, CHUNK)], out_hbm.at[c, pl.ds(row, CHUNK)])


@jax.jit
def _agg(hwp, srcp, dstp):
    k = pl.kernel(
        _agg_body,
        out_type=jax.ShapeDtypeStruct((NC, NP, D), jnp.float32),
        mesh=_mesh,
        scratch_types=[
            pltpu.VMEM((CPT, CHUNK), jnp.int32),
            pltpu.VMEM((CPT, CHUNK), jnp.int32),
            pltpu.VMEM((CHUNK, D), jnp.float32),
            pltpu.VMEM_SHARED((NP, D), jnp.float32),
        ],
    )
    return k(hwp, srcp, dstp)


def _dinv(h0_ref, h1_ref):
    deg = h0_ref[...][:, 0:1] + h1_ref[...][:, 0:1] + 1.0
    return lax.rsqrt(deg)


def _first_body(x_ref, w_ref, h0_ref, h1_ref, o_ref):
    hw = jnp.dot(x_ref[...], w_ref[...], preferred_element_type=jnp.float32)
    o_ref[...] = hw * _dinv(h0_ref, h1_ref)


def _mid_body(p0_ref, p1_ref, hwp_ref, h0_ref, h1_ref, b_ref, w_ref, o_ref):
    dinv = _dinv(h0_ref, h1_ref)
    h = jnp.maximum(dinv * (p0_ref[...] + p1_ref[...] + hwp_ref[...]) + b_ref[...], 0.0)
    o_ref[...] = jnp.dot(h, w_ref[...], preferred_element_type=jnp.float32) * dinv


def _head_body(p0_ref, p1_ref, hwp_ref, h0_ref, h1_ref, b_ref, w1_ref, b1_ref,
               w2_ref, b2_ref, o_ref):
    dinv = _dinv(h0_ref, h1_ref)
    h = jnp.maximum(dinv * (p0_ref[...] + p1_ref[...] + hwp_ref[...]) + b_ref[...], 0.0)
    z = jnp.maximum(jnp.dot(h, w1_ref[...], preferred_element_type=jnp.float32)
                    + b1_ref[...], 0.0)
    o = jnp.dot(z, w2_ref[...], preferred_element_type=jnp.float32) + b2_ref[...]
    m = jnp.max(o, axis=1, keepdims=True)
    ex = jnp.exp(o - m)
    o_ref[...] = (o - m) - jnp.log(jnp.sum(ex, axis=1, keepdims=True))


def _row_spec(width):
    return pl.BlockSpec((RB, width), lambda i: (i, 0))


def _full_spec(r, cdim):
    return pl.BlockSpec((r, cdim), lambda i: (0, 0))


@jax.jit
def _first(xp, W, h0, h1):
    return pl.pallas_call(
        _first_body,
        grid=(NP // RB,),
        in_specs=[_row_spec(D), _full_spec(D, D), _row_spec(16), _row_spec(16)],
        out_specs=_row_spec(D),
        out_shape=jax.ShapeDtypeStruct((NP, D), jnp.float32),
    )(xp, W, h0, h1)


@jax.jit
def _mid(p0, p1, hwp, h0, h1, b, W):
    return pl.pallas_call(
        _mid_body,
        grid=(NP // RB,),
        in_specs=[_row_spec(D), _row_spec(D), _row_spec(D), _row_spec(16),
                  _row_spec(16), _full_spec(1, D), _full_spec(D, D)],
        out_specs=_row_spec(D),
        out_shape=jax.ShapeDtypeStruct((NP, D), jnp.float32),
    )(p0, p1, hwp, h0, h1, b, W)


@jax.jit
def _head(p0, p1, hwp, h0, h1, b, w1, b1, w2, b2):
    return pl.pallas_call(
        _head_body,
        grid=(NP // RB,),
        in_specs=[_row_spec(D), _row_spec(D), _row_spec(D), _row_spec(16),
                  _row_spec(16), _full_spec(1, D), _full_spec(D, D),
                  _full_spec(1, D), _full_spec(D, OUT), _full_spec(1, OUT)],
        out_specs=_row_spec(OUT),
        out_shape=jax.ShapeDtypeStruct((NP, OUT), jnp.float32),
    )(p0, p1, hwp, h0, h1, b, w1, b1, w2, b2)


def kernel(x, edge_index, W0, b0, W1, b1, W2, b2, lin1_W, lin1_b, lin2_W, lin2_b):
    src = edge_index[0]
    dst = edge_index[1]
    srcp = jnp.concatenate(
        [src, jnp.zeros((EP - E,), jnp.int32)]).reshape(NW, CPT, CHUNK)
    dstp = jnp.concatenate(
        [dst, jnp.full((EP - E,), DUMMY, jnp.int32)]).reshape(NW, CPT, CHUNK)
    xp = jnp.zeros((NP, D), jnp.float32).at[:N].set(x)

    hist = _hist(dstp)
    h0, h1 = hist[0], hist[1]

    hwp = _first(xp, W0, h0, h1)
    for b, W in ((b0, W1), (b1, W2)):
        p = _agg(hwp, srcp, dstp)
        hwp = _mid(p[0], p[1], hwp, h0, h1, b.reshape(1, D), W)
    p = _agg(hwp, srcp, dstp)
    out = _head(p[0], p[1], hwp, h0, h1, b2.reshape(1, D),
                lin1_W, lin1_b.reshape(1, D), lin2_W, lin2_b.reshape(1, OUT))
    return out[:N]
